# baseline (speedup 1.0000x reference)
import functools

import jax
import jax.numpy as jnp
from jax.experimental import pallas as pl
from jax.experimental.pallas import tpu as pltpu


def _se_tile(x_ref, w1_ref, w2t_ref, o_ref, *, inv_hw):
    xs = x_ref[...]

    pooled = jnp.sum(xs, axis=(1, 2), dtype=jnp.float32) * inv_hw

    hid = jax.lax.dot_general(pooled, w1_ref[...],
                              (((1,), (1,)), ((), ())),
                              preferred_element_type=jnp.float32)
    hid = jnp.maximum(hid, 0.0)
    gate = jax.lax.dot_general(hid, w2t_ref[...],
                               (((1,), (0,)), ((), ())),
                               preferred_element_type=jnp.float32)
    gate = jax.nn.sigmoid(gate).astype(xs.dtype)

    o_ref[...] = (xs * gate[:, None, None, :]).astype(o_ref.dtype)


def _pick_batch_tile(B, per_batch_bytes, budget_bytes):
    fits = [t for t in range(1, B + 1)
            if B % t == 0 and t * per_batch_bytes <= budget_bytes]
    if not fits:
        return 1
    even = [t for t in fits if (B // t) % 2 == 0]
    return max(even) if even else max(fits)


def kernel(x, w1, w2):
    B, C, H, W = x.shape
    itemsize = jnp.dtype(x.dtype).itemsize
    per_batch = C * H * W * itemsize

    xt = jnp.transpose(x, (0, 2, 3, 1))
    w2t = jnp.transpose(w2)

    tb = _pick_batch_tile(B, per_batch, 2 << 20)
    grid = (B // tb,)

    vmem_limit = 40 << 20

    out = pl.pallas_call(
        functools.partial(_se_tile, inv_hw=float(1.0 / (H * W))),
        out_shape=jax.ShapeDtypeStruct((B, H, W, C), x.dtype),
        grid=grid,
        in_specs=[
            pl.BlockSpec((tb, H, W, C), lambda b: (b, 0, 0, 0)),
            pl.BlockSpec(w1.shape, lambda b: (0, 0)),
            pl.BlockSpec(w2t.shape, lambda b: (0, 0)),
        ],
        out_specs=pl.BlockSpec((tb, H, W, C), lambda b: (b, 0, 0, 0)),
        compiler_params=pltpu.CompilerParams(
            dimension_semantics=("parallel",),
            vmem_limit_bytes=vmem_limit),
    )(xt, w1, w2t)
    return jnp.transpose(out, (0, 3, 1, 2))


# speedup vs baseline: 1.2316x; 1.2316x over previous
import functools

import jax
import jax.numpy as jnp
from jax.experimental import pallas as pl
from jax.experimental.pallas import tpu as pltpu


def _se_tile(x_ref, w1_ref, w2t_ref, o_ref, *, inv_hw):
    xs = x_ref[...]

    pooled = jnp.sum(xs, axis=(1, 2), dtype=jnp.float32) * inv_hw

    hid = jax.lax.dot_general(pooled, w1_ref[...],
                              (((1,), (1,)), ((), ())),
                              preferred_element_type=jnp.float32)
    hid = jnp.maximum(hid, 0.0)
    gate = jax.lax.dot_general(hid, w2t_ref[...],
                               (((1,), (0,)), ((), ())),
                               preferred_element_type=jnp.float32)
    gate = jax.nn.sigmoid(gate).astype(xs.dtype)

    o_ref[...] = (xs * gate[:, None, None, :]).astype(o_ref.dtype)


def _pick_batch_tile(B, per_batch_bytes, budget_bytes):
    fits = [t for t in range(1, B + 1)
            if B % t == 0 and t * per_batch_bytes <= budget_bytes]
    if not fits:
        return 1
    even = [t for t in fits if (B // t) % 2 == 0]
    return max(even) if even else max(fits)


def kernel(x, w1, w2):
    B, C, H, W = x.shape
    itemsize = jnp.dtype(x.dtype).itemsize
    per_batch = C * H * W * itemsize

    xt = jnp.transpose(x, (0, 2, 3, 1))
    w2t = jnp.transpose(w2)

    tb = _pick_batch_tile(B, per_batch, 8 << 20)
    grid = (B // tb,)

    vmem_limit = 40 << 20

    out = pl.pallas_call(
        functools.partial(_se_tile, inv_hw=float(1.0 / (H * W))),
        out_shape=jax.ShapeDtypeStruct((B, H, W, C), x.dtype),
        grid=grid,
        in_specs=[
            pl.BlockSpec((tb, H, W, C), lambda b: (b, 0, 0, 0)),
            pl.BlockSpec(w1.shape, lambda b: (0, 0)),
            pl.BlockSpec(w2t.shape, lambda b: (0, 0)),
        ],
        out_specs=pl.BlockSpec((tb, H, W, C), lambda b: (b, 0, 0, 0)),
        compiler_params=pltpu.CompilerParams(
            dimension_semantics=("parallel",),
            vmem_limit_bytes=vmem_limit),
    )(xt, w1, w2t)
    return jnp.transpose(out, (0, 3, 1, 2))
